# baseline (speedup 1.0000x reference)
import functools

import jax
import jax.numpy as jnp
from jax import lax
from jax.experimental import pallas as pl
from jax.experimental.pallas import tpu as pltpu
from jax.experimental.pallas import tpu_sc as plsc

_N = 10000
_E = 320000
_D = 128
_R = 50
_CUT = 10.0

_BW = 6400
_NBW = _E // _BW


def _wfilter_body(ea_ref, ew_ref, fw1_ref, fb1_ref, fw2_ref, fb2_ref,
                  w_ref):
    ea = ea_ref[...].astype(jnp.bfloat16)
    ew = ew_ref[...]
    c = 0.5 * (jnp.cos((jnp.pi / _CUT) * ew) + 1.0)
    c = jnp.where(ew < _CUT, c, 0.0)
    pad = (-(_BW // _D)) % 8
    cp = jnp.concatenate(
        [c[0], jnp.zeros((pad, _D), jnp.float32)], axis=0)
    ct = jnp.transpose(cp)
    t = jnp.tanh(jnp.dot(ea, fw1_ref[...].astype(jnp.bfloat16),
                         preferred_element_type=jnp.float32) + fb1_ref[...])
    w = jnp.dot(t.astype(jnp.bfloat16),
                fw2_ref[...].astype(jnp.bfloat16),
                preferred_element_type=jnp.float32) + fb2_ref[...]
    for r in range(_BW // _D):
        w_ref[pl.ds(_D * r, _D), :] = (w[_D * r:_D * (r + 1), :]
                                       * ct[:, r:r + 1])


def _wfilter(ea, ew_l, p):
    full = lambda shape: pl.BlockSpec(shape, lambda i: (0, 0))
    return pl.pallas_call(
        _wfilter_body,
        grid=(_NBW,),
        in_specs=[
            pl.BlockSpec((_BW, _R), lambda i: (i, 0)),
            pl.BlockSpec((1, _BW // _D, _D), lambda i: (i, 0, 0)),
            full((_R, _D)), full((1, _D)), full((_D, _D)), full((1, _D)),
        ],
        out_specs=pl.BlockSpec((_BW, _D), lambda i: (i, 0)),
        out_shape=jax.ShapeDtypeStruct((_E, _D), jnp.float32),
    )(ea, ew_l, *p)


def _hproj_body(x_ref, w_ref, h_ref):
    h_ref[...] = jnp.dot(x_ref[...].astype(jnp.bfloat16),
                         w_ref[...].astype(jnp.bfloat16),
                         preferred_element_type=jnp.float32)


def _hproj(x, w):
    return pl.pallas_call(
        _hproj_body,
        out_shape=jax.ShapeDtypeStruct((_N, _D), jnp.float32),
    )(x, w)


_BR = 2000


def _post_body(agg_ref, x_ref, lin2W_ref, lin2b_ref, linW_ref, linb_ref,
               lin1Wn_ref, y_ref, h_ref):
    a = (agg_ref[0] + agg_ref[1]).astype(jnp.bfloat16)
    t = jnp.tanh(jnp.dot(a, lin2W_ref[...].astype(jnp.bfloat16),
                         preferred_element_type=jnp.float32) + lin2b_ref[...])
    y = jnp.dot(t.astype(jnp.bfloat16), linW_ref[...].astype(jnp.bfloat16),
                preferred_element_type=jnp.float32) + linb_ref[...] + x_ref[...]
    y_ref[...] = y
    h_ref[...] = jnp.dot(y.astype(jnp.bfloat16),
                         lin1Wn_ref[...].astype(jnp.bfloat16),
                         preferred_element_type=jnp.float32)


def _post(agg, x, lin2W, lin2b, linW, linb, lin1W_next):
    full = lambda shape: pl.BlockSpec(shape, lambda i: (0, 0))
    return pl.pallas_call(
        _post_body,
        grid=(_N // _BR,),
        in_specs=[
            pl.BlockSpec((2, _BR, _D), lambda i: (0, i, 0)),
            pl.BlockSpec((_BR, _D), lambda i: (i, 0)),
            full((_D, _D)), full((1, _D)), full((_D, _D)), full((1, _D)),
            full((_D, _D)),
        ],
        out_specs=[pl.BlockSpec((_BR, _D), lambda i: (i, 0)),
                   pl.BlockSpec((_BR, _D), lambda i: (i, 0))],
        out_shape=[jax.ShapeDtypeStruct((_N, _D), jnp.float32),
                   jax.ShapeDtypeStruct((_N, _D), jnp.float32)],
    )(agg, x, lin2W, lin2b, linW, linb, lin1W_next)


_NWORK = 32
_NE_W = _E // _NWORK
_K = 40
_NCH = _NE_W // _K
_ZR = 16
_NZC = _N // _ZR
_ZTRIP = -(-_NZC // 16)


def _sc_message_body(h_hbm, w_hbm, src_hbm, dst_hbm, out_hbm,
                     agg_sh,
                     idx_s, idx_d, rows, wrow, zbuf,
                     sem_i, sem_g, sem_s):
    cid = lax.axis_index("c")
    sid = lax.axis_index("s")
    base = (cid * 16 + sid) * _NE_W

    zero = jnp.zeros((16,), jnp.float32)
    for r in range(_ZR):
        for j in range(8):
            zbuf[r, pl.ds(j * 16, 16)] = zero

    def zcp(t, carry):
        c = sid + 16 * t

        @pl.when(c < _NZC)
        def _():
            pltpu.sync_copy(zbuf, agg_sh.at[pl.ds(c * _ZR, _ZR)])

        return carry

    lax.fori_loop(0, _ZTRIP, zcp, 0)
    plsc.subcore_barrier()

    def fetch_idx(g, s):
        eb = base + g * _K
        pltpu.async_copy(src_hbm.at[pl.ds(eb, _K)], idx_s[s], sem_i[s])
        pltpu.async_copy(dst_hbm.at[pl.ds(eb, _K)], idx_d[s], sem_i[s])

    def wait_idx(g, s):
        eb = base + g * _K
        pltpu.make_async_copy(src_hbm.at[pl.ds(eb, _K)], idx_s[s],
                              sem_i[s]).wait()
        pltpu.make_async_copy(dst_hbm.at[pl.ds(eb, _K)], idx_d[s],
                              sem_i[s]).wait()

    def start_gather(g, s):
        eb = base + g * _K
        pltpu.async_copy(h_hbm.at[idx_s[s]], rows[s], sem_g[s])
        pltpu.async_copy(w_hbm.at[pl.ds(eb, _K)], wrow[s], sem_g[s])

    def wait_gather(g, s):
        eb = base + g * _K
        pltpu.make_async_copy(h_hbm.at[idx_s[s]], rows[s], sem_g[s]).wait()
        pltpu.make_async_copy(w_hbm.at[pl.ds(eb, _K)], wrow[s], sem_g[s]).wait()

    def wait_scatter(s):
        pltpu.make_async_copy(rows[s], agg_sh.at[idx_d[s]], sem_s[s]).wait()

    def slot(c, p):
        @pl.when(c <= _NCH - 3)
        def _():

            @pl.when(c >= 1)
            def _():
                wait_scatter((p + 3) % 4)

            wait_idx(c + 2, (p + 2) % 4)
            start_gather(c + 2, (p + 2) % 4)

        @pl.when(jnp.logical_and(c > _NCH - 3, c >= 1))
        def _():
            wait_scatter((p + 3) % 4)

        @pl.when(c <= _NCH - 4)
        def _():
            fetch_idx(c + 3, (p + 3) % 4)

        wait_gather(c, p)

        @plsc.parallel_loop(0, _K, unroll=4)
        def edge(i):
            for j in range(8):
                sl = pl.ds(j * 16, 16)
                rows[p][i, sl] = rows[p][i, sl] * wrow[p][i, sl]
        pltpu.async_copy(rows[p], agg_sh.at[idx_d[p]], sem_s[p], add=True)

    fetch_idx(0, 0)
    fetch_idx(1, 1)
    fetch_idx(2, 2)
    wait_idx(0, 0)
    start_gather(0, 0)
    wait_idx(1, 1)
    start_gather(1, 1)

    def quad(t, carry):
        for p in range(4):
            slot(4 * t + p, p)
        return carry

    lax.fori_loop(0, _NCH // 4, quad, 0)
    for c in range(4 * (_NCH // 4), _NCH):
        slot(jnp.int32(c), c % 4)
    wait_scatter((_NCH - 1) % 4)
    plsc.subcore_barrier()

    def outcp(t, carry):
        c = sid + 16 * t

        @pl.when(c < _NZC)
        def _():
            r0 = c * _ZR
            pltpu.sync_copy(agg_sh.at[pl.ds(r0, _ZR)],
                            out_hbm.at[cid, pl.ds(r0, _ZR)])

        return carry

    lax.fori_loop(0, _ZTRIP, outcp, 0)


def _sc_message(h, w, src, dst):
    mesh = plsc.VectorSubcoreMesh(core_axis_name="c", subcore_axis_name="s")
    fn = pl.kernel(
        _sc_message_body,
        out_type=jax.ShapeDtypeStruct((2, _N, _D), jnp.float32),
        mesh=mesh,
        scratch_types=[
            pltpu.VMEM_SHARED((_N, _D), jnp.float32),
            [pltpu.VMEM((_K,), jnp.int32) for _ in range(4)],
            [pltpu.VMEM((_K,), jnp.int32) for _ in range(4)],
            [pltpu.VMEM((_K, _D), jnp.float32) for _ in range(4)],
            [pltpu.VMEM((_K, _D), jnp.float32) for _ in range(4)],
            pltpu.VMEM((_ZR, _D), jnp.float32),
            [pltpu.SemaphoreType.DMA for _ in range(4)],
            [pltpu.SemaphoreType.DMA for _ in range(4)],
            [pltpu.SemaphoreType.DMA for _ in range(4)],
        ],
    )
    return fn(h, w, src, dst)


def kernel(x, edge_index, edge_weight, edge_attr,
           fW1_0, fb1_0, fW2_0, fb2_0, lin1W_0, lin2W_0, lin2b_0, linW_0,
           linb_0,
           fW1_1, fb1_1, fW2_1, fb2_1, lin1W_1, lin2W_1, lin2b_1, linW_1,
           linb_1):
    src = edge_index[0]
    dst = edge_index[1]
    ew_l = edge_weight.reshape(_NBW, _BW // _D, _D)
    p0 = (fW1_0, fb1_0[None, :], fW2_0, fb2_0[None, :])
    p1 = (fW1_1, fb1_1[None, :], fW2_1, fb2_1[None, :])

    w0 = _wfilter(edge_attr, ew_l, p0)

    h0 = _hproj(x, lin1W_0)
    agg0 = _sc_message(h0, w0, src, dst)
    w1 = _wfilter(edge_attr, ew_l, p1)
    x1, h1 = _post(agg0, x, lin2W_0, lin2b_0[None, :], linW_0,
                   linb_0[None, :], lin1W_1)

    agg1 = _sc_message(h1, w1, src, dst)
    x2, _ = _post(agg1, x1, lin2W_1, lin2b_1[None, :], linW_1,
                  linb_1[None, :], lin1W_1)
    return x2


# speedup vs baseline: 1.0020x; 1.0020x over previous
import functools

import jax
import jax.numpy as jnp
from jax import lax
from jax.experimental import pallas as pl
from jax.experimental.pallas import tpu as pltpu
from jax.experimental.pallas import tpu_sc as plsc

_N = 10000
_E = 320000
_D = 128
_R = 50
_CUT = 10.0

_BW = 6400
_NBW = _E // _BW


def _wfilter_body(ea_ref, ew_ref, fw1_ref, fb1_ref, fw2_ref, fb2_ref,
                  w_ref):
    ea = ea_ref[...].astype(jnp.bfloat16)
    ew = ew_ref[...]
    c = 0.5 * (jnp.cos((jnp.pi / _CUT) * ew) + 1.0)
    c = jnp.where(ew < _CUT, c, 0.0)
    pad = (-(_BW // _D)) % 8
    cp = jnp.concatenate(
        [c[0], jnp.zeros((pad, _D), jnp.float32)], axis=0)
    ct = jnp.transpose(cp)
    t = jnp.tanh(jnp.dot(ea, fw1_ref[...].astype(jnp.bfloat16),
                         preferred_element_type=jnp.float32) + fb1_ref[...])
    w = jnp.dot(t.astype(jnp.bfloat16),
                fw2_ref[...].astype(jnp.bfloat16),
                preferred_element_type=jnp.float32) + fb2_ref[...]
    for r in range(_BW // _D):
        w_ref[pl.ds(_D * r, _D), :] = (w[_D * r:_D * (r + 1), :]
                                       * ct[:, r:r + 1])


def _wfilter(ea, ew_l, p):
    full = lambda shape: pl.BlockSpec(shape, lambda i: (0, 0))
    return pl.pallas_call(
        _wfilter_body,
        grid=(_NBW,),
        in_specs=[
            pl.BlockSpec((_BW, _R), lambda i: (i, 0)),
            pl.BlockSpec((1, _BW // _D, _D), lambda i: (i, 0, 0)),
            full((_R, _D)), full((1, _D)), full((_D, _D)), full((1, _D)),
        ],
        out_specs=pl.BlockSpec((_BW, _D), lambda i: (i, 0)),
        out_shape=jax.ShapeDtypeStruct((_E, _D), jnp.float32),
    )(ea, ew_l, *p)


def _hproj_body(x_ref, w_ref, h_ref):
    h_ref[...] = jnp.dot(x_ref[...].astype(jnp.bfloat16),
                         w_ref[...].astype(jnp.bfloat16),
                         preferred_element_type=jnp.float32)


def _hproj(x, w):
    return pl.pallas_call(
        _hproj_body,
        out_shape=jax.ShapeDtypeStruct((_N, _D), jnp.float32),
    )(x, w)


_BR = 2000


def _post_body(agg_ref, x_ref, lin2W_ref, lin2b_ref, linW_ref, linb_ref,
               lin1Wn_ref, y_ref, h_ref):
    a = (agg_ref[0] + agg_ref[1]).astype(jnp.bfloat16)
    t = jnp.tanh(jnp.dot(a, lin2W_ref[...].astype(jnp.bfloat16),
                         preferred_element_type=jnp.float32) + lin2b_ref[...])
    y = jnp.dot(t.astype(jnp.bfloat16), linW_ref[...].astype(jnp.bfloat16),
                preferred_element_type=jnp.float32) + linb_ref[...] + x_ref[...]
    y_ref[...] = y
    h_ref[...] = jnp.dot(y.astype(jnp.bfloat16),
                         lin1Wn_ref[...].astype(jnp.bfloat16),
                         preferred_element_type=jnp.float32)


def _post(agg, x, lin2W, lin2b, linW, linb, lin1W_next):
    full = lambda shape: pl.BlockSpec(shape, lambda i: (0, 0))
    return pl.pallas_call(
        _post_body,
        grid=(_N // _BR,),
        in_specs=[
            pl.BlockSpec((2, _BR, _D), lambda i: (0, i, 0)),
            pl.BlockSpec((_BR, _D), lambda i: (i, 0)),
            full((_D, _D)), full((1, _D)), full((_D, _D)), full((1, _D)),
            full((_D, _D)),
        ],
        out_specs=[pl.BlockSpec((_BR, _D), lambda i: (i, 0)),
                   pl.BlockSpec((_BR, _D), lambda i: (i, 0))],
        out_shape=[jax.ShapeDtypeStruct((_N, _D), jnp.float32),
                   jax.ShapeDtypeStruct((_N, _D), jnp.float32)],
    )(agg, x, lin2W, lin2b, linW, linb, lin1W_next)


_NWORK = 32
_NE_W = _E // _NWORK
_K = 40
_NCH = _NE_W // _K
_ZR = 16
_NZC = _N // _ZR
_ZTRIP = -(-_NZC // 16)


def _sc_message_body(h_hbm, w_hbm, src_hbm, dst_hbm, out_hbm,
                     agg_sh,
                     idx_s, idx_d, rows, wrow, zbuf,
                     sem_i, sem_g, sem_s):
    cid = lax.axis_index("c")
    sid = lax.axis_index("s")
    base = (cid * 16 + sid) * _NE_W

    zero = jnp.zeros((16,), jnp.float32)
    for r in range(_ZR):
        for j in range(8):
            zbuf[r, pl.ds(j * 16, 16)] = zero

    def zcp(t, carry):
        c = sid + 16 * t

        @pl.when(c < _NZC)
        def _():
            pltpu.sync_copy(zbuf, agg_sh.at[pl.ds(c * _ZR, _ZR)])

        return carry

    lax.fori_loop(0, _ZTRIP, zcp, 0)
    plsc.subcore_barrier()

    def fetch_idx(g, s):
        eb = base + g * _K
        pltpu.async_copy(src_hbm.at[pl.ds(eb, _K)], idx_s[s], sem_i[s])
        pltpu.async_copy(dst_hbm.at[pl.ds(eb, _K)], idx_d[s], sem_i[s])

    def wait_idx(g, s):
        eb = base + g * _K
        pltpu.make_async_copy(src_hbm.at[pl.ds(eb, _K)], idx_s[s],
                              sem_i[s]).wait()
        pltpu.make_async_copy(dst_hbm.at[pl.ds(eb, _K)], idx_d[s],
                              sem_i[s]).wait()

    def start_gather(g, s):
        eb = base + g * _K
        pltpu.async_copy(h_hbm.at[idx_s[s]], rows[s], sem_g[s])
        pltpu.async_copy(w_hbm.at[pl.ds(eb, _K)], wrow[s], sem_g[s])

    def wait_gather(g, s):
        eb = base + g * _K
        pltpu.make_async_copy(h_hbm.at[idx_s[s]], rows[s], sem_g[s]).wait()
        pltpu.make_async_copy(w_hbm.at[pl.ds(eb, _K)], wrow[s], sem_g[s]).wait()

    def wait_scatter(s):
        pltpu.make_async_copy(rows[s], agg_sh.at[idx_d[s]], sem_s[s]).wait()

    def slot(c, p):
        @pl.when(c <= _NCH - 3)
        def _():

            @pl.when(c >= 1)
            def _():
                wait_scatter((p + 3) % 4)

            wait_idx(c + 2, (p + 2) % 4)
            start_gather(c + 2, (p + 2) % 4)

        @pl.when(jnp.logical_and(c > _NCH - 3, c >= 1))
        def _():
            wait_scatter((p + 3) % 4)

        @pl.when(c <= _NCH - 4)
        def _():
            fetch_idx(c + 3, (p + 3) % 4)

        wait_gather(c, p)

        def edge(i, c2):
            for j in range(8):
                sl = pl.ds(j * 16, 16)
                rows[p][i, sl] = rows[p][i, sl] * wrow[p][i, sl]
            return c2

        lax.fori_loop(0, _K, edge, 0)
        pltpu.async_copy(rows[p], agg_sh.at[idx_d[p]], sem_s[p], add=True)

    fetch_idx(0, 0)
    fetch_idx(1, 1)
    fetch_idx(2, 2)
    wait_idx(0, 0)
    start_gather(0, 0)
    wait_idx(1, 1)
    start_gather(1, 1)

    def quad(t, carry):
        for p in range(4):
            slot(4 * t + p, p)
        return carry

    lax.fori_loop(0, _NCH // 4, quad, 0)
    for c in range(4 * (_NCH // 4), _NCH):
        slot(jnp.int32(c), c % 4)
    wait_scatter((_NCH - 1) % 4)
    plsc.subcore_barrier()

    def outcp(t, carry):
        c = sid + 16 * t

        @pl.when(c < _NZC)
        def _():
            r0 = c * _ZR
            pltpu.sync_copy(agg_sh.at[pl.ds(r0, _ZR)],
                            out_hbm.at[cid, pl.ds(r0, _ZR)])

        return carry

    lax.fori_loop(0, _ZTRIP, outcp, 0)


def _sc_message(h, w, src, dst):
    mesh = plsc.VectorSubcoreMesh(core_axis_name="c", subcore_axis_name="s")
    fn = pl.kernel(
        _sc_message_body,
        out_type=jax.ShapeDtypeStruct((2, _N, _D), jnp.float32),
        mesh=mesh,
        scratch_types=[
            pltpu.VMEM_SHARED((_N, _D), jnp.float32),
            [pltpu.VMEM((_K,), jnp.int32) for _ in range(4)],
            [pltpu.VMEM((_K,), jnp.int32) for _ in range(4)],
            [pltpu.VMEM((_K, _D), jnp.float32) for _ in range(4)],
            [pltpu.VMEM((_K, _D), jnp.float32) for _ in range(4)],
            pltpu.VMEM((_ZR, _D), jnp.float32),
            [pltpu.SemaphoreType.DMA for _ in range(4)],
            [pltpu.SemaphoreType.DMA for _ in range(4)],
            [pltpu.SemaphoreType.DMA for _ in range(4)],
        ],
    )
    return fn(h, w, src, dst)


def kernel(x, edge_index, edge_weight, edge_attr,
           fW1_0, fb1_0, fW2_0, fb2_0, lin1W_0, lin2W_0, lin2b_0, linW_0,
           linb_0,
           fW1_1, fb1_1, fW2_1, fb2_1, lin1W_1, lin2W_1, lin2b_1, linW_1,
           linb_1):
    src = edge_index[0]
    dst = edge_index[1]
    ew_l = edge_weight.reshape(_NBW, _BW // _D, _D)
    p0 = (fW1_0, fb1_0[None, :], fW2_0, fb2_0[None, :])
    p1 = (fW1_1, fb1_1[None, :], fW2_1, fb2_1[None, :])

    w0 = _wfilter(edge_attr, ew_l, p0)

    h0 = _hproj(x, lin1W_0)
    agg0 = _sc_message(h0, w0, src, dst)
    w1 = _wfilter(edge_attr, ew_l, p1)
    x1, h1 = _post(agg0, x, lin2W_0, lin2b_0[None, :], linW_0,
                   linb_0[None, :], lin1W_1)

    agg1 = _sc_message(h1, w1, src, dst)
    x2, _ = _post(agg1, x1, lin2W_1, lin2b_1[None, :], linW_1,
                  linb_1[None, :], lin1W_1)
    return x2


# speedup vs baseline: 1.0583x; 1.0562x over previous
import functools

import jax
import jax.numpy as jnp
from jax import lax
from jax.experimental import pallas as pl
from jax.experimental.pallas import tpu as pltpu
from jax.experimental.pallas import tpu_sc as plsc

_N = 10000
_E = 320000
_D = 128
_R = 50
_CUT = 10.0

_BW = 6400
_NBW = _E // _BW


def _wfilter_body(ea_ref, ew_ref, fw1_ref, fb1_ref, fw2_ref, fb2_ref,
                  w_ref):
    ea = ea_ref[...].astype(jnp.bfloat16)
    ew = ew_ref[...]
    c = 0.5 * (jnp.cos((jnp.pi / _CUT) * ew) + 1.0)
    c = jnp.where(ew < _CUT, c, 0.0)
    pad = (-(_BW // _D)) % 8
    cp = jnp.concatenate(
        [c[0], jnp.zeros((pad, _D), jnp.float32)], axis=0)
    ct = jnp.transpose(cp)
    t = jnp.tanh(jnp.dot(ea, fw1_ref[...].astype(jnp.bfloat16),
                         preferred_element_type=jnp.float32) + fb1_ref[...])
    w = jnp.dot(t.astype(jnp.bfloat16),
                fw2_ref[...].astype(jnp.bfloat16),
                preferred_element_type=jnp.float32) + fb2_ref[...]
    for r in range(_BW // _D):
        w_ref[pl.ds(_D * r, _D), :] = (w[_D * r:_D * (r + 1), :]
                                       * ct[:, r:r + 1])


def _wfilter(ea, ew_l, p):
    full = lambda shape: pl.BlockSpec(shape, lambda i: (0, 0))
    return pl.pallas_call(
        _wfilter_body,
        grid=(_NBW,),
        in_specs=[
            pl.BlockSpec((_BW, _R), lambda i: (i, 0)),
            pl.BlockSpec((1, _BW // _D, _D), lambda i: (i, 0, 0)),
            full((_R, _D)), full((1, _D)), full((_D, _D)), full((1, _D)),
        ],
        out_specs=pl.BlockSpec((_BW, _D), lambda i: (i, 0)),
        out_shape=jax.ShapeDtypeStruct((_E, _D), jnp.float32),
    )(ea, ew_l, *p)


def _hproj_body(x_ref, w_ref, h_ref):
    h_ref[...] = jnp.dot(x_ref[...].astype(jnp.bfloat16),
                         w_ref[...].astype(jnp.bfloat16),
                         preferred_element_type=jnp.float32)


def _hproj(x, w):
    return pl.pallas_call(
        _hproj_body,
        out_shape=jax.ShapeDtypeStruct((_N, _D), jnp.float32),
    )(x, w)


_BR = 2000


def _post_body(agg_ref, x_ref, lin2W_ref, lin2b_ref, linW_ref, linb_ref,
               lin1Wn_ref, y_ref, h_ref):
    a = (agg_ref[0] + agg_ref[1]).astype(jnp.bfloat16)
    t = jnp.tanh(jnp.dot(a, lin2W_ref[...].astype(jnp.bfloat16),
                         preferred_element_type=jnp.float32) + lin2b_ref[...])
    y = jnp.dot(t.astype(jnp.bfloat16), linW_ref[...].astype(jnp.bfloat16),
                preferred_element_type=jnp.float32) + linb_ref[...] + x_ref[...]
    y_ref[...] = y
    h_ref[...] = jnp.dot(y.astype(jnp.bfloat16),
                         lin1Wn_ref[...].astype(jnp.bfloat16),
                         preferred_element_type=jnp.float32)


def _post(agg, x, lin2W, lin2b, linW, linb, lin1W_next):
    full = lambda shape: pl.BlockSpec(shape, lambda i: (0, 0))
    return pl.pallas_call(
        _post_body,
        grid=(_N // _BR,),
        in_specs=[
            pl.BlockSpec((2, _BR, _D), lambda i: (0, i, 0)),
            pl.BlockSpec((_BR, _D), lambda i: (i, 0)),
            full((_D, _D)), full((1, _D)), full((_D, _D)), full((1, _D)),
            full((_D, _D)),
        ],
        out_specs=[pl.BlockSpec((_BR, _D), lambda i: (i, 0)),
                   pl.BlockSpec((_BR, _D), lambda i: (i, 0))],
        out_shape=[jax.ShapeDtypeStruct((_N, _D), jnp.float32),
                   jax.ShapeDtypeStruct((_N, _D), jnp.float32)],
    )(agg, x, lin2W, lin2b, linW, linb, lin1W_next)


_NWORK = 32
_NE_W = _E // _NWORK
_K = 40
_NCH = _NE_W // _K
_ZR = 16
_NZC = _N // _ZR
_ZTRIP = -(-_NZC // 16)


def _sc_message_body(h_hbm, w_hbm, src_hbm, dst_hbm, out_hbm,
                     agg_sh,
                     idx_s, idx_d, rows, wrow, zbuf,
                     sem_i, sem_g, sem_s, sem_z):
    cid = lax.axis_index("c")
    sid = lax.axis_index("s")
    base = (cid * 16 + sid) * _NE_W

    zero = jnp.zeros((16,), jnp.float32)
    for r in range(_ZR):
        for j in range(8):
            zbuf[r, pl.ds(j * 16, 16)] = zero

    def zcp(t, carry):
        c = sid + 16 * t

        @pl.when(c < _NZC)
        def _():
            pltpu.async_copy(zbuf, agg_sh.at[pl.ds(c * _ZR, _ZR)], sem_z)

        return carry

    lax.fori_loop(0, _ZTRIP, zcp, 0)

    def zdrain(t, carry):
        c = sid + 16 * t

        @pl.when(c < _NZC)
        def _():
            pltpu.make_async_copy(zbuf, agg_sh.at[pl.ds(c * _ZR, _ZR)],
                                  sem_z).wait()

        return carry

    lax.fori_loop(0, _ZTRIP, zdrain, 0)
    plsc.subcore_barrier()

    def fetch_idx(g, s):
        eb = base + g * _K
        pltpu.async_copy(src_hbm.at[pl.ds(eb, _K)], idx_s[s], sem_i[s])
        pltpu.async_copy(dst_hbm.at[pl.ds(eb, _K)], idx_d[s], sem_i[s])

    def wait_idx(g, s):
        eb = base + g * _K
        pltpu.make_async_copy(src_hbm.at[pl.ds(eb, _K)], idx_s[s],
                              sem_i[s]).wait()
        pltpu.make_async_copy(dst_hbm.at[pl.ds(eb, _K)], idx_d[s],
                              sem_i[s]).wait()

    def start_gather(g, s):
        eb = base + g * _K
        pltpu.async_copy(h_hbm.at[idx_s[s]], rows[s], sem_g[s])
        pltpu.async_copy(w_hbm.at[pl.ds(eb, _K)], wrow[s], sem_g[s])

    def wait_gather(g, s):
        eb = base + g * _K
        pltpu.make_async_copy(h_hbm.at[idx_s[s]], rows[s], sem_g[s]).wait()
        pltpu.make_async_copy(w_hbm.at[pl.ds(eb, _K)], wrow[s], sem_g[s]).wait()

    def wait_scatter(s):
        pltpu.make_async_copy(rows[s], agg_sh.at[idx_d[s]], sem_s[s]).wait()

    def slot(c, p):
        @pl.when(c <= _NCH - 3)
        def _():

            @pl.when(c >= 1)
            def _():
                wait_scatter((p + 3) % 4)

            wait_idx(c + 2, (p + 2) % 4)
            start_gather(c + 2, (p + 2) % 4)

        @pl.when(jnp.logical_and(c > _NCH - 3, c >= 1))
        def _():
            wait_scatter((p + 3) % 4)

        @pl.when(c <= _NCH - 4)
        def _():
            fetch_idx(c + 3, (p + 3) % 4)

        wait_gather(c, p)

        def edge(i, c2):
            for j in range(8):
                sl = pl.ds(j * 16, 16)
                rows[p][i, sl] = rows[p][i, sl] * wrow[p][i, sl]
            return c2

        lax.fori_loop(0, _K, edge, 0)
        pltpu.async_copy(rows[p], agg_sh.at[idx_d[p]], sem_s[p], add=True)

    fetch_idx(0, 0)
    fetch_idx(1, 1)
    fetch_idx(2, 2)
    wait_idx(0, 0)
    start_gather(0, 0)
    wait_idx(1, 1)
    start_gather(1, 1)

    def quad(t, carry):
        for p in range(4):
            slot(4 * t + p, p)
        return carry

    lax.fori_loop(0, _NCH // 4, quad, 0)
    for c in range(4 * (_NCH // 4), _NCH):
        slot(jnp.int32(c), c % 4)
    wait_scatter((_NCH - 1) % 4)
    plsc.subcore_barrier()

    def outcp(t, carry):
        c = sid + 16 * t

        @pl.when(c < _NZC)
        def _():
            r0 = c * _ZR
            pltpu.async_copy(agg_sh.at[pl.ds(r0, _ZR)],
                             out_hbm.at[cid, pl.ds(r0, _ZR)], sem_z)

        return carry

    lax.fori_loop(0, _ZTRIP, outcp, 0)

    def outdrain(t, carry):
        c = sid + 16 * t

        @pl.when(c < _NZC)
        def _():
            r0 = c * _ZR
            pltpu.make_async_copy(agg_sh.at[pl.ds(r0, _ZR)],
                                  out_hbm.at[cid, pl.ds(r0, _ZR)],
                                  sem_z).wait()

        return carry

    lax.fori_loop(0, _ZTRIP, outdrain, 0)


def _sc_message(h, w, src, dst):
    mesh = plsc.VectorSubcoreMesh(core_axis_name="c", subcore_axis_name="s")
    fn = pl.kernel(
        _sc_message_body,
        out_type=jax.ShapeDtypeStruct((2, _N, _D), jnp.float32),
        mesh=mesh,
        scratch_types=[
            pltpu.VMEM_SHARED((_N, _D), jnp.float32),
            [pltpu.VMEM((_K,), jnp.int32) for _ in range(4)],
            [pltpu.VMEM((_K,), jnp.int32) for _ in range(4)],
            [pltpu.VMEM((_K, _D), jnp.float32) for _ in range(4)],
            [pltpu.VMEM((_K, _D), jnp.float32) for _ in range(4)],
            pltpu.VMEM((_ZR, _D), jnp.float32),
            [pltpu.SemaphoreType.DMA for _ in range(4)],
            [pltpu.SemaphoreType.DMA for _ in range(4)],
            [pltpu.SemaphoreType.DMA for _ in range(4)],
            pltpu.SemaphoreType.DMA,
        ],
    )
    return fn(h, w, src, dst)


def kernel(x, edge_index, edge_weight, edge_attr,
           fW1_0, fb1_0, fW2_0, fb2_0, lin1W_0, lin2W_0, lin2b_0, linW_0,
           linb_0,
           fW1_1, fb1_1, fW2_1, fb2_1, lin1W_1, lin2W_1, lin2b_1, linW_1,
           linb_1):
    src = edge_index[0]
    dst = edge_index[1]
    ew_l = edge_weight.reshape(_NBW, _BW // _D, _D)
    p0 = (fW1_0, fb1_0[None, :], fW2_0, fb2_0[None, :])
    p1 = (fW1_1, fb1_1[None, :], fW2_1, fb2_1[None, :])

    w0 = _wfilter(edge_attr, ew_l, p0)

    h0 = _hproj(x, lin1W_0)
    agg0 = _sc_message(h0, w0, src, dst)
    w1 = _wfilter(edge_attr, ew_l, p1)
    x1, h1 = _post(agg0, x, lin2W_0, lin2b_0[None, :], linW_0,
                   linb_0[None, :], lin1W_1)

    agg1 = _sc_message(h1, w1, src, dst)
    x2, _ = _post(agg1, x1, lin2W_1, lin2b_1[None, :], linW_1,
                  linb_1[None, :], lin1W_1)
    return x2


# speedup vs baseline: 1.0583x; 1.0000x over previous
import functools

import jax
import jax.numpy as jnp
from jax import lax
from jax.experimental import pallas as pl
from jax.experimental.pallas import tpu as pltpu
from jax.experimental.pallas import tpu_sc as plsc

_N = 10000
_E = 320000
_D = 128
_R = 50
_CUT = 10.0

_BW = 6400
_NBW = _E // _BW


def _wfilter_body(ea_ref, ew_ref, fw1_ref, fb1_ref, fw2_ref, fb2_ref,
                  w_ref):
    ea = ea_ref[...].astype(jnp.bfloat16)
    ew = ew_ref[...]
    c = 0.5 * (jnp.cos((jnp.pi / _CUT) * ew) + 1.0)
    c = jnp.where(ew < _CUT, c, 0.0)
    pad = (-(_BW // _D)) % 8
    cp = jnp.concatenate(
        [c[0], jnp.zeros((pad, _D), jnp.float32)], axis=0)
    ct = jnp.transpose(cp)
    t = jnp.tanh(jnp.dot(ea, fw1_ref[...].astype(jnp.bfloat16),
                         preferred_element_type=jnp.float32) + fb1_ref[...])
    w = jnp.dot(t.astype(jnp.bfloat16),
                fw2_ref[...].astype(jnp.bfloat16),
                preferred_element_type=jnp.float32) + fb2_ref[...]
    for r in range(_BW // _D):
        w_ref[pl.ds(_D * r, _D), :] = (w[_D * r:_D * (r + 1), :]
                                       * ct[:, r:r + 1])


def _wfilter(ea, ew_l, p):
    full = lambda shape: pl.BlockSpec(shape, lambda i: (0, 0))
    return pl.pallas_call(
        _wfilter_body,
        grid=(_NBW,),
        in_specs=[
            pl.BlockSpec((_BW, _R), lambda i: (i, 0)),
            pl.BlockSpec((1, _BW // _D, _D), lambda i: (i, 0, 0)),
            full((_R, _D)), full((1, _D)), full((_D, _D)), full((1, _D)),
        ],
        out_specs=pl.BlockSpec((_BW, _D), lambda i: (i, 0)),
        out_shape=jax.ShapeDtypeStruct((_E, _D), jnp.float32),
    )(ea, ew_l, *p)


def _hproj_body(x_ref, w_ref, h_ref):
    h_ref[...] = jnp.dot(x_ref[...].astype(jnp.bfloat16),
                         w_ref[...].astype(jnp.bfloat16),
                         preferred_element_type=jnp.float32)


def _hproj(x, w):
    return pl.pallas_call(
        _hproj_body,
        out_shape=jax.ShapeDtypeStruct((_N, _D), jnp.float32),
    )(x, w)


_BR = 2000


def _post_body(agg_ref, x_ref, lin2W_ref, lin2b_ref, linW_ref, linb_ref,
               lin1Wn_ref, y_ref, h_ref):
    a = (agg_ref[0] + agg_ref[1]).astype(jnp.bfloat16)
    t = jnp.tanh(jnp.dot(a, lin2W_ref[...].astype(jnp.bfloat16),
                         preferred_element_type=jnp.float32) + lin2b_ref[...])
    y = jnp.dot(t.astype(jnp.bfloat16), linW_ref[...].astype(jnp.bfloat16),
                preferred_element_type=jnp.float32) + linb_ref[...] + x_ref[...]
    y_ref[...] = y
    h_ref[...] = jnp.dot(y.astype(jnp.bfloat16),
                         lin1Wn_ref[...].astype(jnp.bfloat16),
                         preferred_element_type=jnp.float32)


def _post(agg, x, lin2W, lin2b, linW, linb, lin1W_next):
    full = lambda shape: pl.BlockSpec(shape, lambda i: (0, 0))
    return pl.pallas_call(
        _post_body,
        grid=(_N // _BR,),
        in_specs=[
            pl.BlockSpec((2, _BR, _D), lambda i: (0, i, 0)),
            pl.BlockSpec((_BR, _D), lambda i: (i, 0)),
            full((_D, _D)), full((1, _D)), full((_D, _D)), full((1, _D)),
            full((_D, _D)),
        ],
        out_specs=[pl.BlockSpec((_BR, _D), lambda i: (i, 0)),
                   pl.BlockSpec((_BR, _D), lambda i: (i, 0))],
        out_shape=[jax.ShapeDtypeStruct((_N, _D), jnp.float32),
                   jax.ShapeDtypeStruct((_N, _D), jnp.float32)],
    )(agg, x, lin2W, lin2b, linW, linb, lin1W_next)


_NWORK = 32
_NE_W = _E // _NWORK
_K = 40
_NCH = _NE_W // _K
_ZR = 16
_NZC = _N // _ZR
_ZTRIP = -(-_NZC // 16)


def _sc_message_body(h_hbm, w_hbm, src_hbm, dst_hbm, out_hbm,
                     agg_sh,
                     idx_s, idx_d, rows, wrow, zbuf,
                     sem_i, sem_g, sem_s, sem_z):
    cid = lax.axis_index("c")
    sid = lax.axis_index("s")
    base = (cid * 16 + sid) * _NE_W

    zero = jnp.zeros((16,), jnp.float32)
    for r in range(_ZR):
        for j in range(8):
            zbuf[r, pl.ds(j * 16, 16)] = zero

    def zcp(t, carry):
        c = sid + 16 * t

        @pl.when(c < _NZC)
        def _():
            pltpu.async_copy(zbuf, agg_sh.at[pl.ds(c * _ZR, _ZR)], sem_z)

        return carry

    lax.fori_loop(0, _ZTRIP, zcp, 0)

    def zdrain(t, carry):
        c = sid + 16 * t

        @pl.when(c < _NZC)
        def _():
            pltpu.make_async_copy(zbuf, agg_sh.at[pl.ds(c * _ZR, _ZR)],
                                  sem_z).wait()

        return carry

    lax.fori_loop(0, _ZTRIP, zdrain, 0)
    plsc.subcore_barrier()

    def fetch_idx(g, s):
        eb = base + g * _K
        pltpu.async_copy(src_hbm.at[pl.ds(eb, _K)], idx_s[s], sem_i[s])
        pltpu.async_copy(dst_hbm.at[pl.ds(eb, _K)], idx_d[s], sem_i[s])

    def wait_idx(g, s):
        eb = base + g * _K
        pltpu.make_async_copy(src_hbm.at[pl.ds(eb, _K)], idx_s[s],
                              sem_i[s]).wait()
        pltpu.make_async_copy(dst_hbm.at[pl.ds(eb, _K)], idx_d[s],
                              sem_i[s]).wait()

    def start_gather(g, s):
        eb = base + g * _K
        pltpu.async_copy(h_hbm.at[idx_s[s]], rows[s], sem_g[s])
        pltpu.async_copy(w_hbm.at[pl.ds(eb, _K)], wrow[s], sem_g[s])

    def wait_gather(g, s):
        eb = base + g * _K
        pltpu.make_async_copy(h_hbm.at[idx_s[s]], rows[s], sem_g[s]).wait()
        pltpu.make_async_copy(w_hbm.at[pl.ds(eb, _K)], wrow[s], sem_g[s]).wait()

    def wait_scatter(s):
        pltpu.make_async_copy(rows[s], agg_sh.at[idx_d[s]], sem_s[s]).wait()

    def slot(c, p):
        @pl.when(c <= _NCH - 3)
        def _():

            @pl.when(c >= 1)
            def _():
                wait_scatter((p + 3) % 4)

            wait_idx(c + 2, (p + 2) % 4)
            start_gather(c + 2, (p + 2) % 4)

        @pl.when(jnp.logical_and(c > _NCH - 3, c >= 1))
        def _():
            wait_scatter((p + 3) % 4)

        @pl.when(c <= _NCH - 4)
        def _():
            fetch_idx(c + 3, (p + 3) % 4)

        wait_gather(c, p)

        def edge(i, c2):
            for u in range(4):
                e = 4 * i + u
                for j in range(8):
                    sl = pl.ds(j * 16, 16)
                    rows[p][e, sl] = rows[p][e, sl] * wrow[p][e, sl]
            return c2

        lax.fori_loop(0, _K // 4, edge, 0)
        pltpu.async_copy(rows[p], agg_sh.at[idx_d[p]], sem_s[p], add=True)

    fetch_idx(0, 0)
    fetch_idx(1, 1)
    fetch_idx(2, 2)
    wait_idx(0, 0)
    start_gather(0, 0)
    wait_idx(1, 1)
    start_gather(1, 1)

    def quad(t, carry):
        for p in range(4):
            slot(4 * t + p, p)
        return carry

    lax.fori_loop(0, _NCH // 4, quad, 0)
    for c in range(4 * (_NCH // 4), _NCH):
        slot(jnp.int32(c), c % 4)
    wait_scatter((_NCH - 1) % 4)
    plsc.subcore_barrier()

    def outcp(t, carry):
        c = sid + 16 * t

        @pl.when(c < _NZC)
        def _():
            r0 = c * _ZR
            pltpu.async_copy(agg_sh.at[pl.ds(r0, _ZR)],
                             out_hbm.at[cid, pl.ds(r0, _ZR)], sem_z)

        return carry

    lax.fori_loop(0, _ZTRIP, outcp, 0)

    def outdrain(t, carry):
        c = sid + 16 * t

        @pl.when(c < _NZC)
        def _():
            r0 = c * _ZR
            pltpu.make_async_copy(agg_sh.at[pl.ds(r0, _ZR)],
                                  out_hbm.at[cid, pl.ds(r0, _ZR)],
                                  sem_z).wait()

        return carry

    lax.fori_loop(0, _ZTRIP, outdrain, 0)


def _sc_message(h, w, src, dst):
    mesh = plsc.VectorSubcoreMesh(core_axis_name="c", subcore_axis_name="s")
    fn = pl.kernel(
        _sc_message_body,
        out_type=jax.ShapeDtypeStruct((2, _N, _D), jnp.float32),
        mesh=mesh,
        scratch_types=[
            pltpu.VMEM_SHARED((_N, _D), jnp.float32),
            [pltpu.VMEM((_K,), jnp.int32) for _ in range(4)],
            [pltpu.VMEM((_K,), jnp.int32) for _ in range(4)],
            [pltpu.VMEM((_K, _D), jnp.float32) for _ in range(4)],
            [pltpu.VMEM((_K, _D), jnp.float32) for _ in range(4)],
            pltpu.VMEM((_ZR, _D), jnp.float32),
            [pltpu.SemaphoreType.DMA for _ in range(4)],
            [pltpu.SemaphoreType.DMA for _ in range(4)],
            [pltpu.SemaphoreType.DMA for _ in range(4)],
            pltpu.SemaphoreType.DMA,
        ],
    )
    return fn(h, w, src, dst)


def kernel(x, edge_index, edge_weight, edge_attr,
           fW1_0, fb1_0, fW2_0, fb2_0, lin1W_0, lin2W_0, lin2b_0, linW_0,
           linb_0,
           fW1_1, fb1_1, fW2_1, fb2_1, lin1W_1, lin2W_1, lin2b_1, linW_1,
           linb_1):
    src = edge_index[0]
    dst = edge_index[1]
    ew_l = edge_weight.reshape(_NBW, _BW // _D, _D)
    p0 = (fW1_0, fb1_0[None, :], fW2_0, fb2_0[None, :])
    p1 = (fW1_1, fb1_1[None, :], fW2_1, fb2_1[None, :])

    w0 = _wfilter(edge_attr, ew_l, p0)

    h0 = _hproj(x, lin1W_0)
    agg0 = _sc_message(h0, w0, src, dst)
    w1 = _wfilter(edge_attr, ew_l, p1)
    x1, h1 = _post(agg0, x, lin2W_0, lin2b_0[None, :], linW_0,
                   linb_0[None, :], lin1W_1)

    agg1 = _sc_message(h1, w1, src, dst)
    x2, _ = _post(agg1, x1, lin2W_1, lin2b_1[None, :], linW_1,
                  linb_1[None, :], lin1W_1)
    return x2
